# Initial kernel scaffold; baseline (speedup 1.0000x reference)
import jax
import jax.numpy as jnp
from jax.experimental import pallas as pl


def kernel(context, C0, C1, C2, C3):
    raise NotImplementedError("write your pallas kernel here")


# speedup vs baseline: 16.5515x; 16.5515x over previous
import functools

import jax
import jax.numpy as jnp
from jax import lax
from jax.experimental import pallas as pl
from jax.experimental.pallas import tpu as pltpu
from jax.experimental.pallas import tpu_sc as plsc

HOPS = 3
EMB = 32
B, M, S = 1024, 50, 20
NSEG = B * M
NTABLES = HOPS + 1
NW = 32
SEG_PER_TILE = NSEG // NW
G = 64
CHUNKS = SEG_PER_TILE // G
ROWS = G * S


def _sc_gather_sum(flat_idx, C0, C1, C2, C3):
  mesh = plsc.VectorSubcoreMesh(core_axis_name="c", subcore_axis_name="s")

  @functools.partial(
      pl.kernel,
      out_type=jax.ShapeDtypeStruct((NTABLES * NSEG, EMB), jnp.float32),
      mesh=mesh,
      compiler_params=pltpu.CompilerParams(use_tc_tiling_on_sc=False),
      scratch_types=[
          pltpu.VMEM((ROWS,), jnp.int32),
          pltpu.VMEM((ROWS, EMB), jnp.float32),
          pltpu.VMEM((G, EMB), jnp.float32),
          pltpu.SemaphoreType.DMA,
      ],
  )
  def k(idx_hbm, t0, t1, t2, t3, out_hbm, idx_v, rows_v, e_v, sem):
    wid = lax.axis_index("s") * 2 + lax.axis_index("c")
    tables = [t0, t1, t2, t3]
    for h in range(NTABLES):
      def chunk_body(c, _, h=h):
        seg_base = wid * SEG_PER_TILE + c * G
        row_base = seg_base * S
        pltpu.sync_copy(idx_hbm.at[pl.ds(row_base, ROWS)], idx_v)
        pltpu.async_copy(tables[h].at[idx_v], rows_v, sem).wait()

        def seg_body(g, _):
          r0 = g * S
          lo = jnp.zeros((16,), jnp.float32)
          hi = jnp.zeros((16,), jnp.float32)
          for s in range(S):
            lo = lo + rows_v[r0 + s, 0:16]
            hi = hi + rows_v[r0 + s, 16:32]
          e_v[g, 0:16] = lo
          e_v[g, 16:32] = hi
          return 0

        lax.fori_loop(0, G, seg_body, 0)
        pltpu.sync_copy(e_v, out_hbm.at[pl.ds(h * NSEG + seg_base, G)])
        return 0

      lax.fori_loop(0, CHUNKS, chunk_body, 0)

  return k(flat_idx, C0, C1, C2, C3)


def _tc_attention(e_all):
  BB = 64

  def body(e_ref, o_ref):
    e = e_ref[...]
    q = jnp.zeros((BB, EMB), jnp.float32)
    o2 = None
    for h in range(HOPS):
      mm = e[h]
      p = jnp.sum(mm * q[:, None, :], axis=2)
      p = p - jnp.max(p, axis=1, keepdims=True)
      a = jnp.exp(p)
      a = a / jnp.sum(a, axis=1, keepdims=True)
      c = e[h + 1]
      o2 = jnp.sum(c * a[:, :, None], axis=1)
      q = q + o2
    o_ref[...] = o2

  return pl.pallas_call(
      body,
      grid=(B // BB,),
      in_specs=[pl.BlockSpec((NTABLES, BB, M, EMB), lambda i: (0, i, 0, 0))],
      out_specs=pl.BlockSpec((BB, EMB), lambda i: (i, 0)),
      out_shape=jax.ShapeDtypeStruct((B, EMB), jnp.float32),
  )(e_all)


def kernel(context, C0, C1, C2, C3):
  flat = context.reshape(-1).astype(jnp.int32)
  e = _sc_gather_sum(flat, C0, C1, C2, C3)
  e_all = e.reshape(NTABLES, B, M, EMB)
  return _tc_attention(e_all)


# speedup vs baseline: 23.9597x; 1.4476x over previous
import functools

import jax
import jax.numpy as jnp
from jax import lax
from jax.experimental import pallas as pl
from jax.experimental.pallas import tpu as pltpu
from jax.experimental.pallas import tpu_sc as plsc

HOPS = 3
EMB = 32
B, M, S = 1024, 50, 20
NSEG = B * M
NTABLES = HOPS + 1
NW = 32
B_PER_TILE = B // NW
G = M
ROWS = G * S


def _sc_gather_sum(idx2d, C0, C1, C2, C3):
  mesh = plsc.VectorSubcoreMesh(core_axis_name="c", subcore_axis_name="s")

  @functools.partial(
      pl.kernel,
      out_type=jax.ShapeDtypeStruct((NTABLES * NSEG, EMB), jnp.float32),
      mesh=mesh,
      compiler_params=pltpu.CompilerParams(use_tc_tiling_on_sc=False),
      scratch_types=[
          pltpu.VMEM((B_PER_TILE, ROWS), jnp.int32),
          pltpu.VMEM((2, ROWS, EMB), jnp.float32),
          pltpu.VMEM((2, G, EMB), jnp.float32),
          pltpu.SemaphoreType.DMA,
          pltpu.SemaphoreType.DMA,
          pltpu.SemaphoreType.DMA,
          pltpu.SemaphoreType.DMA,
      ],
  )
  def k(idx_hbm, t0, t1, t2, t3, out_hbm, idx_v, rows_v, e_v, g0, g1, w0, w1):
    wid = lax.axis_index("s") * 2 + lax.axis_index("c")
    tables = [t0, t1, t2, t3]
    gsem = [g0, g1]
    wsem = [w0, w1]

    pltpu.sync_copy(idx_hbm.at[pl.ds(wid * B_PER_TILE, B_PER_TILE)], idx_v)

    def fire(tab, c, par):
      pltpu.async_copy(tab.at[idx_v.at[c]], rows_v.at[par], gsem[par])

    def wait_gather(tab, c, par):
      pltpu.make_async_copy(
          tab.at[idx_v.at[c]], rows_v.at[par], gsem[par]).wait()

    def wait_write(par):
      pltpu.make_async_copy(
          e_v.at[par], out_hbm.at[pl.ds(0, G)], wsem[par]).wait()

    def sum_chunk(par):
      rows = rows_v.at[par]
      e = e_v.at[par]

      def seg_body(g, _):
        r0 = g * S
        lo = jnp.zeros((16,), jnp.float32)
        hi = jnp.zeros((16,), jnp.float32)
        for s in range(S):
          lo = lo + rows[r0 + s, 0:16]
          hi = hi + rows[r0 + s, 16:32]
        e[g, 0:16] = lo
        e[g, 16:32] = hi
        return 0

      lax.fori_loop(0, G, seg_body, 0)

    for h in range(NTABLES):
      tab = tables[h]
      fire(tab, 0, 0)

      def pair_body(cp, _, h=h, tab=tab):
        for par in (0, 1):
          c = cp * 2 + par

          @pl.when(c < B_PER_TILE - 1)
          def _():
            fire(tab, c + 1, 1 - par)

          wait_gather(tab, c, par)

          if h == 0:
            @pl.when(cp > 0)
            def _():
              wait_write(par)
          else:
            wait_write(par)

          sum_chunk(par)
          seg_base = h * NSEG + (wid * B_PER_TILE + c) * G
          pltpu.async_copy(
              e_v.at[par], out_hbm.at[pl.ds(seg_base, G)], wsem[par])
        return 0

      lax.fori_loop(0, B_PER_TILE // 2, pair_body, 0)

    wait_write(0)
    wait_write(1)

  return k(idx2d, C0, C1, C2, C3)


def _tc_attention(e_all):
  BB = 64

  def body(e_ref, o_ref):
    e = e_ref[...]
    q = jnp.zeros((BB, EMB), jnp.float32)
    o2 = None
    for h in range(HOPS):
      mm = e[h]
      p = jnp.sum(mm * q[:, None, :], axis=2)
      p = p - jnp.max(p, axis=1, keepdims=True)
      a = jnp.exp(p)
      a = a / jnp.sum(a, axis=1, keepdims=True)
      c = e[h + 1]
      o2 = jnp.sum(c * a[:, :, None], axis=1)
      q = q + o2
    o_ref[...] = o2

  return pl.pallas_call(
      body,
      grid=(B // BB,),
      in_specs=[pl.BlockSpec((NTABLES, BB, M, EMB), lambda i: (0, i, 0, 0))],
      out_specs=pl.BlockSpec((BB, EMB), lambda i: (i, 0)),
      out_shape=jax.ShapeDtypeStruct((B, EMB), jnp.float32),
  )(e_all)


def kernel(context, C0, C1, C2, C3):
  idx2d = context.reshape(B, M * S).astype(jnp.int32)
  e = _sc_gather_sum(idx2d, C0, C1, C2, C3)
  e_all = e.reshape(NTABLES, B, M, EMB)
  return _tc_attention(e_all)


# speedup vs baseline: 27.3866x; 1.1430x over previous
import functools

import jax
import jax.numpy as jnp
from jax import lax
from jax.experimental import pallas as pl
from jax.experimental.pallas import tpu as pltpu
from jax.experimental.pallas import tpu_sc as plsc

HOPS = 3
EMB = 32
B, M, S = 1024, 50, 20
NSEG = B * M
NTABLES = HOPS + 1
FEMB = NTABLES * EMB
NW = 32
SEG_PER_TILE = NSEG // NW
G = 16
NCHUNK = SEG_PER_TILE // G
ROWS = G * S
NLANE = FEMB // 16


def _sc_gather_sum(idx2d, T4):
  mesh = plsc.VectorSubcoreMesh(core_axis_name="c", subcore_axis_name="s")

  @functools.partial(
      pl.kernel,
      out_type=jax.ShapeDtypeStruct((NSEG, FEMB), jnp.float32),
      mesh=mesh,
      compiler_params=pltpu.CompilerParams(use_tc_tiling_on_sc=False),
      scratch_types=[
          pltpu.VMEM((NCHUNK, ROWS), jnp.int32),
          pltpu.VMEM((2, ROWS, FEMB), jnp.float32),
          pltpu.VMEM((2, G, FEMB), jnp.float32),
          pltpu.SemaphoreType.DMA,
          pltpu.SemaphoreType.DMA,
          pltpu.SemaphoreType.DMA,
          pltpu.SemaphoreType.DMA,
      ],
  )
  def k(idx_hbm, t4, out_hbm, idx_v, rows_v, e_v, g0, g1, w0, w1):
    wid = lax.axis_index("s") * 2 + lax.axis_index("c")
    gsem = [g0, g1]
    wsem = [w0, w1]

    pltpu.sync_copy(idx_hbm.at[wid], idx_v)

    def fire(c, par):
      pltpu.async_copy(t4.at[idx_v.at[c]], rows_v.at[par], gsem[par])

    def wait_gather(c, par):
      pltpu.make_async_copy(
          t4.at[idx_v.at[c]], rows_v.at[par], gsem[par]).wait()

    def wait_write(par):
      pltpu.make_async_copy(
          e_v.at[par], out_hbm.at[pl.ds(0, G)], wsem[par]).wait()

    def sum_chunk(par):
      rows = rows_v.at[par]
      e = e_v.at[par]

      def seg_body(g, _):
        r0 = g * S
        acc = [jnp.zeros((16,), jnp.float32) for _ in range(NLANE)]
        for s in range(S):
          for j in range(NLANE):
            acc[j] = acc[j] + rows[r0 + s, 16 * j:16 * j + 16]
        for j in range(NLANE):
          e[g, 16 * j:16 * j + 16] = acc[j]
        return 0

      lax.fori_loop(0, G, seg_body, 0)

    fire(0, 0)

    def pair_body(cp, _):
      for par in (0, 1):
        c = cp * 2 + par

        @pl.when(c < NCHUNK - 1)
        def _():
          fire(c + 1, 1 - par)

        wait_gather(c, par)

        @pl.when(cp > 0)
        def _():
          wait_write(par)

        sum_chunk(par)
        seg_base = wid * SEG_PER_TILE + c * G
        pltpu.async_copy(
            e_v.at[par], out_hbm.at[pl.ds(seg_base, G)], wsem[par])
      return 0

    lax.fori_loop(0, NCHUNK // 2, pair_body, 0)
    wait_write(0)
    wait_write(1)

  return k(idx2d, T4)


def _tc_attention(e_all):
  BB = 128

  def body(e_ref, o_ref):
    e = e_ref[...]
    q = jnp.zeros((BB, EMB), jnp.float32)
    o2 = None
    for h in range(HOPS):
      mm = e[:, :, EMB * h:EMB * h + EMB]
      p = jnp.sum(mm * q[:, None, :], axis=2)
      p = p - jnp.max(p, axis=1, keepdims=True)
      a = jnp.exp(p)
      a = a / jnp.sum(a, axis=1, keepdims=True)
      c = e[:, :, EMB * (h + 1):EMB * (h + 1) + EMB]
      o2 = jnp.sum(c * a[:, :, None], axis=1)
      q = q + o2
    o_ref[...] = o2

  return pl.pallas_call(
      body,
      grid=(B // BB,),
      in_specs=[pl.BlockSpec((BB, M, FEMB), lambda i: (i, 0, 0))],
      out_specs=pl.BlockSpec((BB, EMB), lambda i: (i, 0)),
      out_shape=jax.ShapeDtypeStruct((B, EMB), jnp.float32),
  )(e_all)


def kernel(context, C0, C1, C2, C3):
  idx2d = context.reshape(NW, NCHUNK, ROWS).astype(jnp.int32)
  T4 = jnp.concatenate([C0, C1, C2, C3], axis=1)
  e4 = _sc_gather_sum(idx2d, T4)
  return _tc_attention(e4.reshape(B, M, FEMB))


# speedup vs baseline: 29.9554x; 1.0938x over previous
import functools

import jax
import jax.numpy as jnp
from jax import lax
from jax.experimental import pallas as pl
from jax.experimental.pallas import tpu as pltpu
from jax.experimental.pallas import tpu_sc as plsc

HOPS = 3
EMB = 32
B, M, S = 1024, 50, 20
NSEG = B * M
NTABLES = HOPS + 1
FEMB = NTABLES * EMB
NW = 32
SEG_PER_TILE = NSEG // NW
G = 16
NCHUNK = SEG_PER_TILE // G
ROWS = G * S
NLANE = FEMB // 16


NSPLIT = 4
BSLICE = B // NSPLIT
SEG_SL = BSLICE * M
SEG_PER_TILE_SL = SEG_SL // NW
NCHUNK_SL = SEG_PER_TILE_SL // G


def _sc_gather_sum(idx2d, T4):
  mesh = plsc.VectorSubcoreMesh(core_axis_name="c", subcore_axis_name="s")

  @functools.partial(
      pl.kernel,
      out_type=jax.ShapeDtypeStruct((SEG_SL, FEMB), jnp.float32),
      mesh=mesh,
      compiler_params=pltpu.CompilerParams(use_tc_tiling_on_sc=False),
      scratch_types=[
          pltpu.VMEM((NCHUNK_SL, ROWS), jnp.int32),
          pltpu.VMEM((2, ROWS, FEMB), jnp.float32),
          pltpu.VMEM((2, G, FEMB), jnp.float32),
          pltpu.SemaphoreType.DMA,
          pltpu.SemaphoreType.DMA,
          pltpu.SemaphoreType.DMA,
          pltpu.SemaphoreType.DMA,
      ],
  )
  def k(idx_hbm, t4, out_hbm, idx_v, rows_v, e_v, g0, g1, w0, w1):
    wid = lax.axis_index("s") * 2 + lax.axis_index("c")
    gsem = [g0, g1]
    wsem = [w0, w1]

    pltpu.sync_copy(idx_hbm.at[wid], idx_v)

    def fire(c, par):
      pltpu.async_copy(t4.at[idx_v.at[c]], rows_v.at[par], gsem[par])

    def wait_gather(c, par):
      pltpu.make_async_copy(
          t4.at[idx_v.at[c]], rows_v.at[par], gsem[par]).wait()

    def wait_write(par):
      pltpu.make_async_copy(
          e_v.at[par], out_hbm.at[pl.ds(0, G)], wsem[par]).wait()

    def sum_chunk(par):
      rows = rows_v.at[par]
      e = e_v.at[par]

      def seg_body(g, _):
        r0 = g * S
        acc = [jnp.zeros((16,), jnp.float32) for _ in range(NLANE)]
        for s in range(S):
          for j in range(NLANE):
            acc[j] = acc[j] + rows[r0 + s, 16 * j:16 * j + 16]
        for j in range(NLANE):
          e[g, 16 * j:16 * j + 16] = acc[j]
        return 0

      lax.fori_loop(0, G, seg_body, 0)

    fire(0, 0)

    def pair_body(cp, _):
      for par in (0, 1):
        c = cp * 2 + par

        @pl.when(c < NCHUNK_SL - 1)
        def _():
          fire(c + 1, 1 - par)

        wait_gather(c, par)

        @pl.when(cp > 0)
        def _():
          wait_write(par)

        sum_chunk(par)
        seg_base = wid * SEG_PER_TILE_SL + c * G
        pltpu.async_copy(
            e_v.at[par], out_hbm.at[pl.ds(seg_base, G)], wsem[par])
      return 0

    lax.fori_loop(0, NCHUNK_SL // 2, pair_body, 0)
    if NCHUNK_SL % 2:
      c = NCHUNK_SL - 1
      wait_gather(c, 0)
      wait_write(0)
      sum_chunk(0)
      seg_base = wid * SEG_PER_TILE_SL + c * G
      pltpu.async_copy(
          e_v.at[0], out_hbm.at[pl.ds(seg_base, G)], wsem[0])
    wait_write(0)
    wait_write(1)

  return k(idx2d, T4)


def _tc_attention(e_all):
  BB = 128

  def body(e_ref, o_ref):
    e = e_ref[...]
    q = jnp.zeros((BB, EMB), jnp.float32)
    o2 = None
    for h in range(HOPS):
      mm = e[:, :, EMB * h:EMB * h + EMB]
      p = jnp.sum(mm * q[:, None, :], axis=2)
      p = p - jnp.max(p, axis=1, keepdims=True)
      a = jnp.exp(p)
      a = a / jnp.sum(a, axis=1, keepdims=True)
      c = e[:, :, EMB * (h + 1):EMB * (h + 1) + EMB]
      o2 = jnp.sum(c * a[:, :, None], axis=1)
      q = q + o2
    o_ref[...] = o2

  return pl.pallas_call(
      body,
      grid=(BSLICE // BB,),
      in_specs=[pl.BlockSpec((BB, M, FEMB), lambda i: (i, 0, 0))],
      out_specs=pl.BlockSpec((BB, EMB), lambda i: (i, 0)),
      out_shape=jax.ShapeDtypeStruct((BSLICE, EMB), jnp.float32),
  )(e_all)


def kernel(context, C0, C1, C2, C3):
  idx_all = context.reshape(NSPLIT, NW, NCHUNK_SL, ROWS).astype(jnp.int32)
  T4 = jnp.concatenate([C0, C1, C2, C3], axis=1)
  outs = []
  for s in range(NSPLIT):
    e4 = _sc_gather_sum(idx_all[s], T4)
    outs.append(_tc_attention(e4.reshape(BSLICE, M, FEMB)))
  return jnp.concatenate(outs, axis=0)
